# baseline (speedup 1.0000x reference)
import functools

import jax
import jax.numpy as jnp
from jax import lax
from jax.experimental import pallas as pl
from jax.experimental.pallas import tpu as pltpu
from jax.experimental.pallas import tpu_sc as plsc

N = 10000
E = 320000
D = 128
NC = 2
NS = 16
TILES = NC * NS
NPAD = 10240
ROWS_PER_TILE = NPAD // NS
EPT = E // TILES
GI = 128
NB = 80
EPT_PAD = NB * GI
PADE = EPT_PAD - EPT
ZROWS = ROWS_PER_TILE

_MESH = plsc.VectorSubcoreMesh(
    core_axis_name="c", subcore_axis_name="s", num_cores=NC, num_subcores=NS)


@functools.partial(
    pl.kernel,
    out_type=jax.ShapeDtypeStruct((NC, NPAD), jnp.float32),
    mesh=_MESH,
    scratch_types=[
        pltpu.VMEM((NB, GI), jnp.int32),
        pltpu.VMEM((GI,), jnp.float32),
        pltpu.VMEM_SHARED((NPAD,), jnp.float32),
    ],
)
def _deg_call(dstw_hbm, zeros1_hbm, ones_hbm, out_hbm, idx_v, ones_v, acc_sh):
    c = lax.axis_index("c")
    s = lax.axis_index("s")
    wid = c * NS + s
    seg = s * ROWS_PER_TILE
    pltpu.sync_copy(zeros1_hbm.at[pl.ds(seg, ROWS_PER_TILE)],
                    acc_sh.at[pl.ds(seg, ROWS_PER_TILE)])
    pltpu.sync_copy(ones_hbm, ones_v)
    pltpu.sync_copy(dstw_hbm.at[wid], idx_v)
    plsc.subcore_barrier()

    def body(j, carry):
        pltpu.sync_copy(ones_v, acc_sh.at[idx_v.at[j]], add=True)
        return carry

    lax.fori_loop(0, NB, body, 0)
    plsc.subcore_barrier()
    pltpu.sync_copy(acc_sh.at[pl.ds(seg, ROWS_PER_TILE)],
                    out_hbm.at[c, pl.ds(seg, ROWS_PER_TILE)])


@functools.partial(
    pl.kernel,
    out_type=jax.ShapeDtypeStruct((NC, NPAD, D), jnp.float32),
    mesh=_MESH,
    scratch_types=[
        pltpu.VMEM((NB, GI), jnp.int32),
        pltpu.VMEM((NB, GI), jnp.int32),
        pltpu.VMEM((GI, D), jnp.float32),
        pltpu.VMEM_SHARED((NPAD, D), jnp.float32),
        pltpu.SemaphoreType.DMA,
    ],
)
def _spmm_call(hs_hbm, srcw_hbm, dstw_hbm, zeros_hbm, out_hbm,
               src_v, dst_v, rows_v, acc_sh, gsem):
    c = lax.axis_index("c")
    s = lax.axis_index("s")
    wid = c * NS + s
    seg = s * ROWS_PER_TILE
    pltpu.sync_copy(zeros_hbm, acc_sh.at[pl.ds(seg, ROWS_PER_TILE)])
    pltpu.sync_copy(srcw_hbm.at[wid], src_v)
    pltpu.sync_copy(dstw_hbm.at[wid], dst_v)
    plsc.subcore_barrier()

    @plsc.parallel_loop(0, NB)
    def body(j):
        pltpu.async_copy(hs_hbm.at[src_v.at[j]], rows_v, gsem).wait()
        pltpu.sync_copy(rows_v, acc_sh.at[dst_v.at[j]], add=True)
    plsc.subcore_barrier()
    pltpu.sync_copy(acc_sh.at[pl.ds(seg, ROWS_PER_TILE)],
                    out_hbm.at[c, pl.ds(seg, ROWS_PER_TILE)])


def _tc1_body(x_ref, w_ref, degp_ref, hs_ref, dinv_ref):
    degp = degp_ref[...]
    dinv = lax.rsqrt(degp[:, 0:1] + degp[:, 1:2] + 1.0)
    h = jnp.dot(x_ref[...], w_ref[...], preferred_element_type=jnp.float32)
    hs_ref[...] = h * dinv[:N]
    dinv_ref[...] = dinv


def _tc2_body(acc_ref, hs1_ref, dinv_ref, b1_ref, g_ref, bb_ref, w2_ref,
              out_ref):
    dinv = dinv_ref[...][:N]
    t = (acc_ref[0, :N, :] + acc_ref[1, :N, :] + hs1_ref[...]) * dinv \
        + b1_ref[...]
    mean = jnp.mean(t, axis=-1, keepdims=True)
    cen = t - mean
    var = jnp.mean(cen * cen, axis=-1, keepdims=True)
    y = cen * lax.rsqrt(var + 1e-5) * g_ref[...] + bb_ref[...]
    y = jnp.maximum(y, 0.0)
    out_ref[...] = jnp.dot(y, w2_ref[...],
                           preferred_element_type=jnp.float32) * dinv


def _tc3_body(acc_ref, hs2_ref, dinv_ref, b2_ref, out_ref):
    dinv = dinv_ref[...][:N]
    out_ref[...] = (acc_ref[0, :N, :] + acc_ref[1, :N, :] + hs2_ref[...]) \
        * dinv + b2_ref[...]


def kernel(x, edge_index, W1, b1, ln_g, ln_b, W2, b2):
    src = edge_index[0]
    dst = edge_index[1]
    srcw = jnp.pad(src.reshape(TILES, EPT),
                   ((0, 0), (0, PADE))).reshape(TILES, NB, GI)
    pad_dst = (N + jnp.arange(TILES * PADE, dtype=jnp.int32)
               % (NPAD - N)).reshape(TILES, PADE)
    dstw = jnp.concatenate([dst.reshape(TILES, EPT), pad_dst],
                           axis=1).reshape(TILES, NB, GI)

    zeros1 = jnp.zeros((NPAD,), jnp.float32)
    zeros2 = jnp.zeros((ZROWS, D), jnp.float32)
    ones = jnp.ones((GI,), jnp.float32)

    degp = _deg_call(dstw, zeros1, ones)
    degp_t = degp.T

    hs1, dinv = pl.pallas_call(
        _tc1_body,
        out_shape=[jax.ShapeDtypeStruct((N, D), jnp.float32),
                   jax.ShapeDtypeStruct((NPAD, 1), jnp.float32)],
    )(x, W1, degp_t)

    acc1 = _spmm_call(hs1, srcw, dstw, zeros2)

    hs2 = pl.pallas_call(
        _tc2_body,
        out_shape=jax.ShapeDtypeStruct((N, D), jnp.float32),
    )(acc1, hs1, dinv, b1.reshape(1, D), ln_g.reshape(1, D),
      ln_b.reshape(1, D), W2)

    acc2 = _spmm_call(hs2, srcw, dstw, zeros2)

    out = pl.pallas_call(
        _tc3_body,
        out_shape=jax.ShapeDtypeStruct((N, D), jnp.float32),
    )(acc2, hs2, dinv, b2.reshape(1, D))
    return out


# speedup vs baseline: 1.4505x; 1.4505x over previous
import functools

import jax
import jax.numpy as jnp
from jax import lax
from jax.experimental import pallas as pl
from jax.experimental.pallas import tpu as pltpu
from jax.experimental.pallas import tpu_sc as plsc

N = 10000
E = 320000
D = 128
NC = 2
NS = 16
TILES = NC * NS
NPAD = 10240
ROWS_PER_TILE = NPAD // NS
EPT = E // TILES
GI = 128
NB = 79
EPT_PAD = NB * GI
PADE = EPT_PAD - EPT
ZROWS = ROWS_PER_TILE

_MESH = plsc.VectorSubcoreMesh(
    core_axis_name="c", subcore_axis_name="s", num_cores=NC, num_subcores=NS)


@functools.partial(
    pl.kernel,
    out_type=jax.ShapeDtypeStruct((NC, NPAD), jnp.float32),
    mesh=_MESH,
    scratch_types=[
        pltpu.VMEM((NB, GI), jnp.int32),
        pltpu.VMEM((GI,), jnp.float32),
        pltpu.VMEM_SHARED((NPAD,), jnp.float32),
    ],
)
def _deg_call(dstw_hbm, zeros1_hbm, ones_hbm, out_hbm, idx_v, ones_v, acc_sh):
    c = lax.axis_index("c")
    s = lax.axis_index("s")
    wid = c * NS + s
    seg = s * ROWS_PER_TILE
    pltpu.sync_copy(zeros1_hbm.at[pl.ds(seg, ROWS_PER_TILE)],
                    acc_sh.at[pl.ds(seg, ROWS_PER_TILE)])
    pltpu.sync_copy(ones_hbm, ones_v)
    pltpu.sync_copy(dstw_hbm.at[wid], idx_v)
    plsc.subcore_barrier()

    def body(j, carry):
        pltpu.sync_copy(ones_v, acc_sh.at[idx_v.at[j]], add=True)
        return carry

    lax.fori_loop(0, NB, body, 0)
    plsc.subcore_barrier()
    pltpu.sync_copy(acc_sh.at[pl.ds(seg, ROWS_PER_TILE)],
                    out_hbm.at[c, pl.ds(seg, ROWS_PER_TILE)])


@functools.partial(
    pl.kernel,
    out_type=jax.ShapeDtypeStruct((NC, NPAD, D), jnp.float32),
    mesh=_MESH,
    scratch_types=[
        pltpu.VMEM((NB, GI), jnp.int32),
        pltpu.VMEM((NB, GI), jnp.int32),
        pltpu.VMEM((GI, D), jnp.float32),
        pltpu.VMEM_SHARED((NPAD, D), jnp.float32),
        pltpu.SemaphoreType.DMA,
    ],
)
def _spmm_call(hs_hbm, srcw_hbm, dstw_hbm, zeros_hbm, out_hbm,
               src_v, dst_v, rows_v, acc_sh, gsem):
    c = lax.axis_index("c")
    s = lax.axis_index("s")
    wid = c * NS + s
    seg = s * ROWS_PER_TILE
    pltpu.sync_copy(zeros_hbm, acc_sh.at[pl.ds(seg, ROWS_PER_TILE)])
    pltpu.sync_copy(srcw_hbm.at[wid], src_v)
    pltpu.sync_copy(dstw_hbm.at[wid], dst_v)
    plsc.subcore_barrier()

    def body(j, carry):
        pltpu.async_copy(hs_hbm.at[src_v.at[j]], rows_v, gsem).wait()
        pltpu.sync_copy(rows_v, acc_sh.at[dst_v.at[j]], add=True)
        return carry

    lax.fori_loop(0, NB, body, 0)
    plsc.subcore_barrier()
    pltpu.sync_copy(acc_sh.at[pl.ds(seg, ROWS_PER_TILE)],
                    out_hbm.at[c, pl.ds(seg, ROWS_PER_TILE)])


def _tc1_body(x_ref, w_ref, degp_ref, hs_ref, dinv_ref):
    degp = degp_ref[...]
    dinv = lax.rsqrt(degp[:, 0:1] + degp[:, 1:2] + 1.0)
    h = jnp.dot(x_ref[...], w_ref[...], preferred_element_type=jnp.float32)
    hs_ref[...] = h * dinv[:N]
    dinv_ref[...] = dinv


def _tc2_body(acc_ref, hs1_ref, dinv_ref, b1_ref, g_ref, bb_ref, w2_ref,
              out_ref):
    dinv = dinv_ref[...][:N]
    t = (acc_ref[0, :N, :] + acc_ref[1, :N, :] + hs1_ref[...]) * dinv \
        + b1_ref[...]
    mean = jnp.mean(t, axis=-1, keepdims=True)
    cen = t - mean
    var = jnp.mean(cen * cen, axis=-1, keepdims=True)
    y = cen * lax.rsqrt(var + 1e-5) * g_ref[...] + bb_ref[...]
    y = jnp.maximum(y, 0.0)
    out_ref[...] = jnp.dot(y, w2_ref[...],
                           preferred_element_type=jnp.float32) * dinv


def _tc3_body(acc_ref, hs2_ref, dinv_ref, b2_ref, out_ref):
    dinv = dinv_ref[...][:N]
    out_ref[...] = (acc_ref[0, :N, :] + acc_ref[1, :N, :] + hs2_ref[...]) \
        * dinv + b2_ref[...]


def kernel(x, edge_index, W1, b1, ln_g, ln_b, W2, b2):
    src = edge_index[0]
    dst = edge_index[1]
    srcw = jnp.pad(src.reshape(TILES, EPT),
                   ((0, 0), (0, PADE))).reshape(TILES, NB, GI)
    pad_dst = (N + jnp.arange(TILES * PADE, dtype=jnp.int32)
               % (NPAD - N)).reshape(TILES, PADE)
    dstw = jnp.concatenate([dst.reshape(TILES, EPT), pad_dst],
                           axis=1).reshape(TILES, NB, GI)

    zeros1 = jnp.zeros((NPAD,), jnp.float32)
    zeros2 = jnp.zeros((ZROWS, D), jnp.float32)
    ones = jnp.ones((GI,), jnp.float32)

    degp = _deg_call(dstw, zeros1, ones)
    degp_t = degp.T

    hs1, dinv = pl.pallas_call(
        _tc1_body,
        out_shape=[jax.ShapeDtypeStruct((N, D), jnp.float32),
                   jax.ShapeDtypeStruct((NPAD, 1), jnp.float32)],
    )(x, W1, degp_t)

    acc1 = _spmm_call(hs1, srcw, dstw, zeros2)

    hs2 = pl.pallas_call(
        _tc2_body,
        out_shape=jax.ShapeDtypeStruct((N, D), jnp.float32),
    )(acc1, hs1, dinv, b1.reshape(1, D), ln_g.reshape(1, D),
      ln_b.reshape(1, D), W2)

    acc2 = _spmm_call(hs2, srcw, dstw, zeros2)

    out = pl.pallas_call(
        _tc3_body,
        out_shape=jax.ShapeDtypeStruct((N, D), jnp.float32),
    )(acc2, hs2, dinv, b2.reshape(1, D))
    return out
